# baseline (speedup 1.0000x reference)
import functools

import numpy as np

import jax
import jax.numpy as jnp
from jax.experimental import pallas as pl
from jax.experimental.pallas import tpu as pltpu

_C = 19
_F = 256
_STEPS = 2
_K = 8192


def _upsample_matrix(h):
    h2 = 2 * h
    fx = (np.arange(h2) + 0.5) / 2.0 - 0.5
    a = np.arange(-1, h + 1)
    w = np.maximum(0.0, 1.0 - np.abs(fx[:, None] - a[None, :]))
    ac = np.clip(a, 0, h - 1)
    u = np.zeros((h2, h), np.float32)
    for r in range(h2):
        for j in range(len(a)):
            u[r, ac[j]] += w[r, j]
    return u


def _up_unc_body(sem_ref, uh_ref, uwt_ref, out_ref, unc_ref):
    h2 = unc_ref.shape[1]
    w2 = unc_ref.shape[2]
    m1 = jnp.full((h2, w2), -jnp.inf, jnp.float32)
    m2 = jnp.full((h2, w2), -jnp.inf, jnp.float32)
    for c in range(_C):
        t = jnp.dot(uh_ref[...], sem_ref[0, c], preferred_element_type=jnp.float32, precision=jax.lax.Precision.HIGHEST)
        o = jnp.dot(t, uwt_ref[...], preferred_element_type=jnp.float32, precision=jax.lax.Precision.HIGHEST)
        out_ref[0, c] = o
        m2 = jnp.maximum(m2, jnp.minimum(m1, o))
        m1 = jnp.maximum(m1, o)
    unc_ref[0] = m2 - m1


def _upsample_unc(sem, uh, uwt):
    n, c, h, w = sem.shape
    h2, w2 = 2 * h, 2 * w
    return pl.pallas_call(
        _up_unc_body,
        grid=(n,),
        in_specs=[
            pl.BlockSpec((1, c, h, w), lambda i: (i, 0, 0, 0)),
            pl.BlockSpec((h2, h), lambda i: (0, 0)),
            pl.BlockSpec((w, w2), lambda i: (0, 0)),
        ],
        out_specs=[
            pl.BlockSpec((1, c, h2, w2), lambda i: (i, 0, 0, 0)),
            pl.BlockSpec((1, h2, w2), lambda i: (i, 0, 0)),
        ],
        out_shape=[
            jax.ShapeDtypeStruct((n, c, h2, w2), jnp.float32),
            jax.ShapeDtypeStruct((n, h2, w2), jnp.float32),
        ],
    )(sem, uh, uwt)


def _mlp_body(fine_ref, coarse_ref, w1_ref, b1_ref, w2_ref, b2_ref,
              w3_ref, b3_ref, wp_ref, bp_ref, out_ref):
    fine = fine_ref[0]
    coarse = coarse_ref[0]
    x = jnp.concatenate([fine, coarse], axis=0)
    for w_ref, b_ref in ((w1_ref, b1_ref), (w2_ref, b2_ref), (w3_ref, b3_ref)):
        h = jnp.maximum(
            jnp.dot(w_ref[...], x, preferred_element_type=jnp.float32)
            + b_ref[...], 0.0)
        x = jnp.concatenate([h, coarse], axis=0)
    out_ref[0] = (jnp.dot(wp_ref[...], x, preferred_element_type=jnp.float32)
                  + bp_ref[...])


def _point_head(fine, coarse, w1, b1, w2, b2, w3, b3, wp, bp):
    n, _, r = fine.shape
    rb = 2048
    grid = (n, r // rb)
    wb = [w1, b1[:, None], w2, b2[:, None], w3, b3[:, None], wp, bp[:, None]]
    wspecs = [pl.BlockSpec(a.shape, lambda i, j: (0, 0)) for a in wb]
    return pl.pallas_call(
        _mlp_body,
        grid=grid,
        in_specs=[
            pl.BlockSpec((1, _F, rb), lambda i, j: (i, 0, j)),
            pl.BlockSpec((1, _C, rb), lambda i, j: (i, 0, j)),
        ] + wspecs,
        out_specs=pl.BlockSpec((1, _C, rb), lambda i, j: (i, 0, j)),
        out_shape=jax.ShapeDtypeStruct((n, _C, r), jnp.float32),
    )(fine, coarse, w1, b1[:, None], w2, b2[:, None], w3, b3[:, None],
      wp, bp[:, None])


def _corners(idx, w2, ratio, feat_w):
    ix = idx % w2
    iy = idx // w2
    sc = 2 * ratio
    fxn = 2 * ix - (ratio - 1)
    fyn = 2 * iy - (ratio - 1)
    x0 = fxn // sc
    y0 = fyn // sc
    wx1 = (fxn - x0 * sc).astype(jnp.float32) / sc
    wy1 = (fyn - y0 * sc).astype(jnp.float32) / sc
    return x0, y0, wx1, wy1


def _sample_and_swsum(feat, x0, y0, wx1, wy1):
    n, c, h, w = feat.shape
    acc = None
    sw = None
    for dy in (0, 1):
        for dx in (0, 1):
            xi = x0 + dx
            yi = y0 + dy
            valid = ((xi >= 0) & (xi <= w - 1) & (yi >= 0) & (yi <= h - 1))
            wgt = (jnp.where(dx == 1, wx1, 1.0 - wx1)
                   * jnp.where(dy == 1, wy1, 1.0 - wy1)
                   * valid.astype(jnp.float32))
            xc = jnp.clip(xi, 0, w - 1)
            yc = jnp.clip(yi, 0, h - 1)
            vals = jax.vmap(lambda f, yy, xx: f[:, yy, xx])(feat, yc, xc)
            term = vals * wgt[:, None, :]
            acc = term if acc is None else acc + term
            sw = wgt if sw is None else sw + wgt
    return acc, sw


def kernel(features, w_coarse, b_coarse, w_fine, b_fine, w1, b1, w2, b2, w3, b3, wp, bp):
    n, cf, h0, w0 = features.shape

    coarse_logits = (jnp.einsum('nchw,kc->nkhw', features, w_coarse)
                     + b_coarse[None, :, None, None])
    low_level = (jnp.einsum('nchw,kc->nkhw', features, w_fine)
                 + b_fine[None, :, None, None])

    uh1 = jnp.asarray(_upsample_matrix(h0))
    uw1t = jnp.asarray(_upsample_matrix(w0).T)
    uh2 = jnp.asarray(_upsample_matrix(2 * h0))
    uw2t = jnp.asarray(_upsample_matrix(2 * w0).T)

    sem = coarse_logits
    for step in range(_STEPS):
        uh, uwt = (uh1, uw1t) if step == 0 else (uh2, uw2t)
        sem_up, unc = _upsample_unc(sem, uh, uwt)
        nn, cc, hu, wu = sem_up.shape
        flat_unc = unc.reshape(n, hu * wu)
        _, idx = jax.lax.top_k(flat_unc, _K)
        ratio = 2 ** (step + 1)
        x0, y0, wx1, wy1 = _corners(idx, wu, ratio, w0)
        fine, _ = _sample_and_swsum(low_level, x0, y0, wx1, wy1)
        coarse_f, _ = _sample_and_swsum(coarse_logits, x0, y0, wx1, wy1)
        plog = _point_head(fine, coarse_f, w1, b1, w2, b2, w3, b3, wp, bp)
        flat = sem_up.reshape(n, cc, hu * wu)
        flat = jax.vmap(lambda f, i, v: f.at[:, i].set(v))(flat, idx, plog)
        sem = flat.reshape(n, cc, hu, wu)
    return sem
